# baseline (speedup 1.0000x reference)
import functools

import jax
import jax.numpy as jnp
from jax.experimental import pallas as pl
from jax.experimental.pallas import tpu as pltpu

_ROWS = 32
_N = 1000000
_BLK = 8192

_KS0 = 0
_KS1 = 42
_KS2 = _KS0 ^ _KS1 ^ 0x1BD11BDA

_ROT_A = (13, 15, 26, 6)
_ROT_B = (17, 29, 16, 24)


def _rotl(x, r):
    return (x << jnp.uint32(r)) | (x >> jnp.uint32(32 - r))


def _four_rounds(x0, x1, rots):
    for r in rots:
        x0 = x0 + x1
        x1 = _rotl(x1, r)
        x1 = x1 ^ x0
    return x0, x1


def _threefry_bits(counts):
    ks0 = jnp.uint32(_KS0)
    ks1 = jnp.uint32(_KS1)
    ks2 = jnp.uint32(_KS2)
    x0 = jnp.zeros_like(counts)
    x1 = counts + ks1
    x0, x1 = _four_rounds(x0, x1, _ROT_A)
    x0, x1 = x0 + ks1, x1 + (ks2 + jnp.uint32(1))
    x0, x1 = _four_rounds(x0, x1, _ROT_B)
    x0, x1 = x0 + ks2, x1 + (ks0 + jnp.uint32(2))
    x0, x1 = _four_rounds(x0, x1, _ROT_A)
    x0, x1 = x0 + ks0, x1 + (ks1 + jnp.uint32(3))
    x0, x1 = _four_rounds(x0, x1, _ROT_B)
    x0, x1 = x0 + ks1, x1 + (ks2 + jnp.uint32(4))
    x0, x1 = _four_rounds(x0, x1, _ROT_A)
    x0, x1 = x0 + ks2, x1 + (ks0 + jnp.uint32(5))
    return x0 ^ x1


def _sample_kernel(preds_ref, val_ref, idx_ref):
    j = pl.program_id(0)
    col0 = (j * _BLK).astype(jnp.uint32)
    row = jax.lax.broadcasted_iota(jnp.uint32, (_ROWS, _BLK), 0)
    col = jax.lax.broadcasted_iota(jnp.uint32, (_ROWS, _BLK), 1)
    gcol = col + col0
    counts = row * jnp.uint32(_N) + gcol
    bits = _threefry_bits(counts)

    tiny = jnp.float32(jnp.finfo(jnp.float32).tiny)
    fb = (bits >> jnp.uint32(9)) | jnp.uint32(0x3F800000)
    u = jax.lax.bitcast_convert_type(fb, jnp.float32) - jnp.float32(1.0)
    u = jnp.maximum(u * (jnp.float32(1.0) - tiny) + tiny, tiny)
    g = -jnp.log(-jnp.log(u))
    z = jnp.log(preds_ref[...]) + g
    z = jnp.where(gcol < jnp.uint32(_N), z, -jnp.inf)

    bm = jnp.max(z, axis=1, keepdims=True)
    gcol_i = gcol.astype(jnp.int32)
    bi = jnp.min(jnp.where(z == bm, gcol_i, jnp.int32(_N)), axis=1,
                 keepdims=True)

    @pl.when(j == 0)
    def _():
        val_ref[...] = bm
        idx_ref[...] = bi

    @pl.when(j != 0)
    def _():
        better = bm > val_ref[...]
        val_ref[...] = jnp.where(better, bm, val_ref[...])
        idx_ref[...] = jnp.where(better, bi, idx_ref[...])


@functools.partial(jax.jit)
def kernel(preds):
    nblk = pl.cdiv(_N, _BLK)
    _, idx = pl.pallas_call(
        _sample_kernel,
        grid=(nblk,),
        in_specs=[pl.BlockSpec((_ROWS, _BLK), lambda j: (0, j))],
        out_specs=[
            pl.BlockSpec((_ROWS, 1), lambda j: (0, 0)),
            pl.BlockSpec((_ROWS, 1), lambda j: (0, 0)),
        ],
        out_shape=[
            jax.ShapeDtypeStruct((_ROWS, 1), jnp.float32),
            jax.ShapeDtypeStruct((_ROWS, 1), jnp.int32),
        ],
        compiler_params=pltpu.CompilerParams(
            dimension_semantics=("arbitrary",),
        ),
    )(preds)
    return idx.reshape(_ROWS)
